# baseline (speedup 1.0000x reference)
import functools

import jax
import jax.numpy as jnp
from jax import lax
from jax.experimental import pallas as pl
from jax.experimental.pallas import tpu as pltpu
from jax.experimental.pallas import tpu_sc as plsc

BATCH = 16384

_info = plsc.get_sparse_core_info()
_NC, _NS = _info.num_cores, _info.num_subcores
_NW = _NC * _NS
_BPW = BATCH // _NW
_CHUNK = 512
_NCHUNK = _BPW // _CHUNK


def _sc_kernel(uidx_hbm, midx_hbm, skill_hbm, diff_hbm, out_hbm,
               uidx_v, midx_v, s_v, d_v, o_v,
               sem_u, sem_m, sem_o, gsems):
    wid = lax.axis_index("s") * _NC + lax.axis_index("c")
    base = wid * _BPW

    cp_u = pltpu.async_copy(uidx_hbm.at[pl.ds(base, _BPW)], uidx_v, sem_u)
    cp_m = pltpu.async_copy(midx_hbm.at[pl.ds(base, _BPW)], midx_v, sem_m)

    skill_flat = skill_hbm.at[0]
    diff_flat = diff_hbm.at[0]
    g_u, g_m = [], []
    cp_u.wait()
    for j in range(_NCHUNK):
        sl = pl.ds(j * _CHUNK, _CHUNK)
        g_u.append(pltpu.async_copy(skill_flat.at[uidx_v.at[sl]], s_v.at[sl], gsems.at[j]))
    cp_m.wait()
    for j in range(_NCHUNK):
        sl = pl.ds(j * _CHUNK, _CHUNK)
        g_m.append(pltpu.async_copy(diff_flat.at[midx_v.at[sl]], d_v.at[sl], gsems.at[j]))

    out_cps = []
    for j in range(_NCHUNK):
        g_u[j].wait()
        g_m[j].wait()
        for i in range(_CHUNK // 16):
            sl = pl.ds(j * _CHUNK + i * 16, 16)
            x = s_v[sl] - d_v[sl]
            o_v[sl] = 1.0 / (1.0 + jnp.exp(-x))
        sl = pl.ds(j * _CHUNK, _CHUNK)
        out_cps.append(pltpu.async_copy(
            o_v.at[sl], out_hbm.at[pl.ds(base + j * _CHUNK, _CHUNK)], sem_o))
    for cp in out_cps:
        cp.wait()


@jax.jit
def kernel(user_idx, map_idx, user_skill, map_diff):
    skill2 = user_skill.reshape(1, -1)
    diff2 = map_diff.reshape(1, -1)
    mesh = plsc.VectorSubcoreMesh(core_axis_name="c", subcore_axis_name="s")
    run = functools.partial(
        pl.kernel,
        mesh=mesh,
        out_type=jax.ShapeDtypeStruct((BATCH,), jnp.float32),
        scratch_types=[
            pltpu.VMEM((_BPW,), jnp.int32),
            pltpu.VMEM((_BPW,), jnp.int32),
            pltpu.VMEM((_BPW,), jnp.float32),
            pltpu.VMEM((_BPW,), jnp.float32),
            pltpu.VMEM((_BPW,), jnp.float32),
            pltpu.SemaphoreType.DMA,
            pltpu.SemaphoreType.DMA,
            pltpu.SemaphoreType.DMA,
            pltpu.SemaphoreType.DMA((_NCHUNK,)),
        ],
    )(_sc_kernel)
    return run(user_idx, map_idx, skill2, diff2)


# speedup vs baseline: 1.0189x; 1.0189x over previous
import functools

import jax
import jax.numpy as jnp
from jax import lax
from jax.experimental import pallas as pl
from jax.experimental.pallas import tpu as pltpu
from jax.experimental.pallas import tpu_sc as plsc

BATCH = 16384

_info = plsc.get_sparse_core_info()
_NC, _NS = _info.num_cores, _info.num_subcores
_NW = _NC * _NS
_BPW = BATCH // _NW
_CHUNK = 256
_NCHUNK = _BPW // _CHUNK


def _sc_kernel(uidx_hbm, midx_hbm, skill_hbm, diff_hbm, out_hbm,
               uidx_v, midx_v, s_v, d_v, o_v,
               sem_u, sem_m, sem_o, gsems):
    wid = lax.axis_index("s") * _NC + lax.axis_index("c")
    base = wid * _BPW

    cp_u = pltpu.async_copy(uidx_hbm.at[pl.ds(base, _BPW)], uidx_v, sem_u)
    cp_m = pltpu.async_copy(midx_hbm.at[pl.ds(base, _BPW)], midx_v, sem_m)

    skill_flat = skill_hbm.at[0]
    diff_flat = diff_hbm.at[0]
    g_u, g_m = [], []
    cp_u.wait()
    for j in range(_NCHUNK):
        sl = pl.ds(j * _CHUNK, _CHUNK)
        g_u.append(pltpu.async_copy(skill_flat.at[uidx_v.at[sl]], s_v.at[sl], gsems.at[j]))
    cp_m.wait()
    for j in range(_NCHUNK):
        sl = pl.ds(j * _CHUNK, _CHUNK)
        g_m.append(pltpu.async_copy(diff_flat.at[midx_v.at[sl]], d_v.at[sl], gsems.at[j]))

    def _sig16(i, j):
        sl = pl.ds(j * _CHUNK + i * 16, 16)
        x = s_v[sl] - d_v[sl]
        o_v[sl] = 1.0 / (1.0 + jnp.exp(-x))
        return j

    out_cps = []
    for j in range(_NCHUNK):
        g_u[j].wait()
        g_m[j].wait()
        lax.fori_loop(0, _CHUNK // 16, _sig16, j, unroll=4)
        sl = pl.ds(j * _CHUNK, _CHUNK)
        out_cps.append(pltpu.async_copy(
            o_v.at[sl], out_hbm.at[pl.ds(base + j * _CHUNK, _CHUNK)], sem_o))
    for cp in out_cps:
        cp.wait()


@jax.jit
def kernel(user_idx, map_idx, user_skill, map_diff):
    skill2 = user_skill.reshape(1, -1)
    diff2 = map_diff.reshape(1, -1)
    mesh = plsc.VectorSubcoreMesh(core_axis_name="c", subcore_axis_name="s")
    run = functools.partial(
        pl.kernel,
        mesh=mesh,
        out_type=jax.ShapeDtypeStruct((BATCH,), jnp.float32),
        scratch_types=[
            pltpu.VMEM((_BPW,), jnp.int32),
            pltpu.VMEM((_BPW,), jnp.int32),
            pltpu.VMEM((_BPW,), jnp.float32),
            pltpu.VMEM((_BPW,), jnp.float32),
            pltpu.VMEM((_BPW,), jnp.float32),
            pltpu.SemaphoreType.DMA,
            pltpu.SemaphoreType.DMA,
            pltpu.SemaphoreType.DMA,
            pltpu.SemaphoreType.DMA((_NCHUNK,)),
        ],
    )(_sc_kernel)
    return run(user_idx, map_idx, skill2, diff2)


# speedup vs baseline: 1.0207x; 1.0017x over previous
import functools

import jax
import jax.numpy as jnp
from jax import lax
from jax.experimental import pallas as pl
from jax.experimental.pallas import tpu as pltpu
from jax.experimental.pallas import tpu_sc as plsc

BATCH = 16384

_info = plsc.get_sparse_core_info()
_NC, _NS = _info.num_cores, _info.num_subcores
_NW = _NC * _NS
_BPW = BATCH // _NW
_CHUNK = 256
_NCHUNK = _BPW // _CHUNK


def _sc_kernel(uidx_hbm, midx_hbm, skill_hbm, diff_hbm, out_hbm,
               uidx_v, midx_v, s_v, d_v, o_v,
               sem_u, sem_m, sem_o, gsems):
    wid = lax.axis_index("s") * _NC + lax.axis_index("c")
    base = wid * _BPW

    cp_u, cp_m = [], []
    for j in range(_NCHUNK):
        sl = pl.ds(j * _CHUNK, _CHUNK)
        cp_u.append(pltpu.async_copy(
            uidx_hbm.at[pl.ds(base + j * _CHUNK, _CHUNK)], uidx_v.at[sl], sem_u.at[j]))
        cp_m.append(pltpu.async_copy(
            midx_hbm.at[pl.ds(base + j * _CHUNK, _CHUNK)], midx_v.at[sl], sem_m.at[j]))

    skill_flat = skill_hbm.at[0]
    diff_flat = diff_hbm.at[0]
    g_u, g_m = [None] * _NCHUNK, [None] * _NCHUNK
    for j in range(_NCHUNK):
        sl = pl.ds(j * _CHUNK, _CHUNK)
        cp_u[j].wait()
        g_u[j] = pltpu.async_copy(skill_flat.at[uidx_v.at[sl]], s_v.at[sl], gsems.at[j])
        cp_m[j].wait()
        g_m[j] = pltpu.async_copy(diff_flat.at[midx_v.at[sl]], d_v.at[sl], gsems.at[j])

    def _sig16(i, j):
        sl = pl.ds(j * _CHUNK + i * 16, 16)
        x = s_v[sl] - d_v[sl]
        o_v[sl] = 1.0 / (1.0 + jnp.exp(-x))
        return j

    out_cps = []
    for j in range(_NCHUNK):
        g_u[j].wait()
        g_m[j].wait()
        lax.fori_loop(0, _CHUNK // 16, _sig16, j, unroll=4)
        sl = pl.ds(j * _CHUNK, _CHUNK)
        out_cps.append(pltpu.async_copy(
            o_v.at[sl], out_hbm.at[pl.ds(base + j * _CHUNK, _CHUNK)], sem_o))
    for cp in out_cps:
        cp.wait()


@jax.jit
def kernel(user_idx, map_idx, user_skill, map_diff):
    skill2 = user_skill.reshape(1, -1)
    diff2 = map_diff.reshape(1, -1)
    mesh = plsc.VectorSubcoreMesh(core_axis_name="c", subcore_axis_name="s")
    run = functools.partial(
        pl.kernel,
        mesh=mesh,
        out_type=jax.ShapeDtypeStruct((BATCH,), jnp.float32),
        scratch_types=[
            pltpu.VMEM((_BPW,), jnp.int32),
            pltpu.VMEM((_BPW,), jnp.int32),
            pltpu.VMEM((_BPW,), jnp.float32),
            pltpu.VMEM((_BPW,), jnp.float32),
            pltpu.VMEM((_BPW,), jnp.float32),
            pltpu.SemaphoreType.DMA((_NCHUNK,)),
            pltpu.SemaphoreType.DMA((_NCHUNK,)),
            pltpu.SemaphoreType.DMA,
            pltpu.SemaphoreType.DMA((_NCHUNK,)),
        ],
    )(_sc_kernel)
    return run(user_idx, map_idx, skill2, diff2)
